# baseline (speedup 1.0000x reference)
import jax
import jax.numpy as jnp
from jax import lax
from jax.experimental import pallas as pl
from jax.experimental.pallas import tpu as pltpu
from jax.experimental.pallas import tpu_sc as plsc

NUM_USERS = 1000000
EXER_N = 100000
BATCH = 16384
DIM = 16
_NC = 2
_NS = 16
_NW = _NC * _NS
_RPW = BATCH // _NW
_G = _RPW // 16


def _mirt_body(stu_ref, exer_ref, th_ref, al_ref, be_ref, out_ref,
               sidx, eidx, tgi, agi, th, al, be, ov, sem_t, sem_a, sem_b):
    wid = lax.axis_index("s") * _NC + lax.axis_index("c")
    base = wid * _RPW
    pltpu.sync_copy(stu_ref.at[pl.ds(base, _RPW)], sidx)
    pltpu.sync_copy(exer_ref.at[pl.ds(base, _RPW)], eidx)

    def idx_body(k, carry):
        ko = pl.multiple_of(k * 16, 16)
        sv = sidx[pl.ds(ko, 16)]
        ev = eidx[pl.ds(ko, 16)]
        for c in range(DIM):
            po = pl.multiple_of(k * 256 + c * 16, 16)
            tgi[pl.ds(po, 16)] = sv + (c * NUM_USERS)
            agi[pl.ds(po, 16)] = ev + (c * EXER_N)
        return carry

    lax.fori_loop(0, _G, idx_body, 0)

    c_th = pltpu.async_copy(th_ref.at[tgi], th, sem_t)
    c_al = pltpu.async_copy(al_ref.at[agi], al, sem_a)
    c_be = pltpu.async_copy(be_ref.at[eidx], be, sem_b)
    c_th.wait()
    c_al.wait()
    c_be.wait()

    def dot_body(k, carry):
        ko = pl.multiple_of(k * 16, 16)
        acc = jnp.zeros((16,), jnp.float32)
        for c in range(DIM):
            po = pl.multiple_of(k * 256 + c * 16, 16)
            acc = acc + th[pl.ds(po, 16)] * al[pl.ds(po, 16)]
        x = acc - be[pl.ds(ko, 16)]
        ov[pl.ds(ko, 16)] = 1.0 / (1.0 + jnp.exp(-x))
        return carry

    lax.fori_loop(0, _G, dot_body, 0)
    pltpu.sync_copy(ov, out_ref.at[pl.ds(base, _RPW)])


def kernel(stu_id, exer_id, kn_emb, thetas, alphas, betas):
    del kn_emb
    th_flat = thetas.T.reshape(-1)
    al_flat = alphas.T.reshape(-1)
    be_flat = betas.reshape(-1)
    mesh = plsc.VectorSubcoreMesh(core_axis_name="c", subcore_axis_name="s",
                                  num_cores=_NC, num_subcores=_NS)
    return pl.kernel(
        _mirt_body,
        out_type=jax.ShapeDtypeStruct((BATCH,), jnp.float32),
        mesh=mesh,
        compiler_params=pltpu.CompilerParams(needs_layout_passes=False),
        scratch_types=[
            pltpu.VMEM((_RPW,), jnp.int32),
            pltpu.VMEM((_RPW,), jnp.int32),
            pltpu.VMEM((_RPW * DIM,), jnp.int32),
            pltpu.VMEM((_RPW * DIM,), jnp.int32),
            pltpu.VMEM((_RPW * DIM,), jnp.float32),
            pltpu.VMEM((_RPW * DIM,), jnp.float32),
            pltpu.VMEM((_RPW,), jnp.float32),
            pltpu.VMEM((_RPW,), jnp.float32),
            pltpu.SemaphoreType.DMA,
            pltpu.SemaphoreType.DMA,
            pltpu.SemaphoreType.DMA,
        ],
    )(stu_id, exer_id, th_flat, al_flat, be_flat)


# speedup vs baseline: 4.0471x; 4.0471x over previous
import jax
import jax.numpy as jnp
from jax import lax
from jax.experimental import pallas as pl
from jax.experimental.pallas import tpu as pltpu
from jax.experimental.pallas import tpu_sc as plsc

BATCH = 16384
DIM = 16
_NC = 2
_NS = 16
_NW = _NC * _NS
_RPW = BATCH // _NW
_CH = 128
_NCHUNK = _RPW // _CH


def _mirt_body(stu_ref, exer_ref, thetas_ref, alg_ref, be_ref, out_ref,
               sidx, eidx, agi, th0, th1, ag, be, ov,
               sem_a, sem_b, sem_t0, sem_t1):
    wid = lax.axis_index("s") * _NC + lax.axis_index("c")
    base = wid * _RPW
    pltpu.sync_copy(stu_ref.at[pl.ds(base, _RPW)], sidx)
    pltpu.sync_copy(exer_ref.at[pl.ds(base, _RPW)], eidx)

    def agi_body(k, carry):
        ko = pl.multiple_of(k * 16, 16)
        agi[pl.ds(ko, 16)] = lax.shift_right_logical(eidx[pl.ds(ko, 16)], 3)
        return carry

    lax.fori_loop(0, _RPW // 16, agi_body, 0)
    c_al = pltpu.async_copy(alg_ref.at[agi], ag, sem_a)
    c_be = pltpu.async_copy(be_ref.at[eidx], be, sem_b)

    lane = lax.iota(jnp.int32, 16)
    ths = [th0, th1]
    sems = [sem_t0, sem_t1]

    def fire_chunk(s, buf, sem):
        def fire(k, carry):
            o = pl.multiple_of(s * _CH + k * 16, 16)
            sv = sidx[pl.ds(o, 16)]
            for l in range(16):
                pltpu.async_copy(thetas_ref.at[pl.ds(sv[l], 1), :],
                                 buf.at[pl.ds(k * 16 + l, 1), :], sem)
            return carry

        lax.fori_loop(0, _CH // 16, fire, 0)

    fire_chunk(0, ths[0], sems[0])

    for s in range(_NCHUNK):
        if s + 1 < _NCHUNK:
            fire_chunk(s + 1, ths[(s + 1) % 2], sems[(s + 1) % 2])
        pltpu.make_async_copy(thetas_ref.at[pl.ds(0, _CH), :], ths[s % 2],
                              sems[s % 2]).wait()
        if s == 0:
            c_al.wait()
            c_be.wait()

        th = ths[s % 2]

        def chunk_body(k, carry):
            o = pl.multiple_of(s * _CH + k * 16, 16)
            ev = eidx[pl.ds(o, 16)]
            trows = k * 16 + lane
            arows = s * _CH + k * 16 + lane
            acb = lax.shift_left(jnp.bitwise_and(ev, 7), 4)
            acc = jnp.zeros((16,), jnp.float32)
            for c in range(DIM):
                cv = jnp.full((16,), c, jnp.int32)
                acc = acc + plsc.load_gather(th, [trows, cv]) * \
                    plsc.load_gather(ag, [arows, acb + c])
            x = acc - be[pl.ds(o, 16)]
            ov[pl.ds(o, 16)] = 1.0 / (1.0 + jnp.exp(-x))
            return carry

        lax.fori_loop(0, _CH // 16, chunk_body, 0)

    pltpu.sync_copy(ov, out_ref.at[pl.ds(base, _RPW)])


def kernel(stu_id, exer_id, kn_emb, thetas, alphas, betas):
    del kn_emb
    alg = alphas.reshape(-1, 128)
    be_flat = betas.reshape(-1)
    mesh = plsc.VectorSubcoreMesh(core_axis_name="c", subcore_axis_name="s",
                                  num_cores=_NC, num_subcores=_NS)
    return pl.kernel(
        _mirt_body,
        out_type=jax.ShapeDtypeStruct((BATCH,), jnp.float32),
        mesh=mesh,
        compiler_params=pltpu.CompilerParams(needs_layout_passes=False),
        scratch_types=[
            pltpu.VMEM((_RPW,), jnp.int32),
            pltpu.VMEM((_RPW,), jnp.int32),
            pltpu.VMEM((_RPW,), jnp.int32),
            pltpu.VMEM((_CH, DIM), jnp.float32),
            pltpu.VMEM((_CH, DIM), jnp.float32),
            pltpu.VMEM((_RPW, 128), jnp.float32),
            pltpu.VMEM((_RPW,), jnp.float32),
            pltpu.VMEM((_RPW,), jnp.float32),
            pltpu.SemaphoreType.DMA,
            pltpu.SemaphoreType.DMA,
            pltpu.SemaphoreType.DMA,
            pltpu.SemaphoreType.DMA,
        ],
    )(stu_id, exer_id, thetas, alg, be_flat)


# speedup vs baseline: 4.1086x; 1.0152x over previous
import jax
import jax.numpy as jnp
from jax import lax
from jax.experimental import pallas as pl
from jax.experimental.pallas import tpu as pltpu
from jax.experimental.pallas import tpu_sc as plsc

BATCH = 16384
DIM = 16
_NC = 2
_NS = 16
_NW = _NC * _NS
_RPW = BATCH // _NW
_CH = 128
_NCHUNK = _RPW // _CH


def _mirt_body(stu_ref, exer_ref, thetas_ref, alphas_ref, betas_ref, out_ref,
               sidx, eidx, th, al, be, ov, sem_t, sem_a, sem_b):
    wid = lax.axis_index("s") * _NC + lax.axis_index("c")
    base = wid * _RPW
    pltpu.sync_copy(stu_ref.at[pl.ds(base, _RPW)], sidx)
    pltpu.sync_copy(exer_ref.at[pl.ds(base, _RPW)], eidx)

    c_be = pltpu.async_copy(betas_ref.at[eidx], be, sem_b)

    lane = lax.iota(jnp.int32, 16)

    for s in range(_NCHUNK):
        def fire(k, carry):
            o = pl.multiple_of(s * _CH + k * 16, 16)
            sv = sidx[pl.ds(o, 16)]
            ev = eidx[pl.ds(o, 16)]
            for l in range(16):
                dst = pl.ds(k * 16 + l, 1)
                pltpu.async_copy(thetas_ref.at[pl.ds(sv[l], 1), :],
                                 th.at[dst, :], sem_t)
                pltpu.async_copy(alphas_ref.at[pl.ds(ev[l], 1), :],
                                 al.at[dst, :], sem_a)
            return carry

        lax.fori_loop(0, _CH // 16, fire, 0)
        pltpu.make_async_copy(thetas_ref.at[pl.ds(0, _CH), :], th,
                              sem_t).wait()
        pltpu.make_async_copy(alphas_ref.at[pl.ds(0, _CH), :], al,
                              sem_a).wait()
        if s == 0:
            c_be.wait()

        def chunk_body(k, carry):
            o = pl.multiple_of(s * _CH + k * 16, 16)
            rows = k * 16 + lane
            acc = jnp.zeros((16,), jnp.float32)
            for c in range(DIM):
                cv = jnp.full((16,), c, jnp.int32)
                acc = acc + plsc.load_gather(th, [rows, cv]) * \
                    plsc.load_gather(al, [rows, cv])
            x = acc - be[pl.ds(o, 16)]
            ov[pl.ds(o, 16)] = 1.0 / (1.0 + jnp.exp(-x))
            return carry

        lax.fori_loop(0, _CH // 16, chunk_body, 0)

    pltpu.sync_copy(ov, out_ref.at[pl.ds(base, _RPW)])


def kernel(stu_id, exer_id, kn_emb, thetas, alphas, betas):
    del kn_emb
    mesh = plsc.VectorSubcoreMesh(core_axis_name="c", subcore_axis_name="s",
                                  num_cores=_NC, num_subcores=_NS)
    return pl.kernel(
        _mirt_body,
        out_type=jax.ShapeDtypeStruct((BATCH,), jnp.float32),
        mesh=mesh,
        compiler_params=pltpu.CompilerParams(needs_layout_passes=False),
        scratch_types=[
            pltpu.VMEM((_RPW,), jnp.int32),
            pltpu.VMEM((_RPW,), jnp.int32),
            pltpu.VMEM((_CH, DIM), jnp.float32),
            pltpu.VMEM((_CH, DIM), jnp.float32),
            pltpu.VMEM((_RPW,), jnp.float32),
            pltpu.VMEM((_RPW,), jnp.float32),
            pltpu.SemaphoreType.DMA,
            pltpu.SemaphoreType.DMA,
            pltpu.SemaphoreType.DMA,
        ],
    )(stu_id, exer_id, thetas, alphas, betas.reshape(-1))
